# baseline (speedup 1.0000x reference)
import jax
import jax.numpy as jnp
from jax.experimental import pallas as pl
from jax.experimental.pallas import tpu as pltpu

K_REAL = 100000
K_PAD = 102400
NG = 800
GPC = 80
CK = GPC * 128
NKC = NG // GPC
D = 128
BQ = 128
TOPK = 30
MW = TOPK + GPC
NEG = -1e30
BIGID = 2**30


def _topk_kernel(q_ref, k_ref, vals_ref, ids_ref,
                 cand_ref, gm_ref, gid_ref):
    ck = pl.program_id(1)

    q = q_ref[...]
    qn = q / (jnp.sqrt(jnp.sum(q * q, axis=-1, keepdims=True)) + 1e-12)
    sims = jax.lax.dot_general(
        qn, k_ref[...], (((1,), (1,)), ((), ())),
        preferred_element_type=jnp.float32)
    sims3 = sims.reshape(BQ, 128, GPC)

    l_io = jax.lax.broadcasted_iota(jnp.int32, (BQ, 128, GPC), 1)
    g_io = jax.lax.broadcasted_iota(jnp.int32, (BQ, 128, GPC), 2)
    orig = (ck * GPC + g_io) * 128 + l_io
    sims3 = jnp.where(orig < K_REAL, sims3, NEG)

    cm = jnp.max(sims3, axis=1)

    @pl.when(ck == 0)
    def _init():
        gm_ref[...] = jnp.full((BQ, 32), NEG, jnp.float32)
        gid_ref[...] = jnp.full((BQ, 32), BIGID, jnp.int32)
        cand_ref[...] = jnp.full((BQ, 128, TOPK), NEG, jnp.float32)

    mv = jnp.concatenate([gm_ref[:, :TOPK], cm], axis=-1)
    mi = jnp.concatenate(
        [gid_ref[:, :TOPK],
         ck * GPC + jax.lax.broadcasted_iota(jnp.int32, (BQ, GPC), 1)],
        axis=-1)
    pos_io = jax.lax.broadcasted_iota(jnp.int32, (BQ, MW), 1)
    ji = jax.lax.broadcasted_iota(jnp.int32, (BQ, 32), 1)

    def mrg_body(i, carry):
        v, gm30, gid30, pos30 = carry
        m = jnp.max(v, axis=-1)
        sid = jnp.min(jnp.where(v == m[:, None], mi, BIGID), axis=-1)
        hit = (v == m[:, None]) & (mi == sid[:, None])
        p = jnp.min(jnp.where(hit, pos_io, MW), axis=-1)
        v = jnp.where(hit, NEG, v)
        gm30 = jnp.where(ji == i, m[:, None], gm30)
        gid30 = jnp.where(ji == i, sid[:, None], gid30)
        pos30 = jnp.where(ji == i, p[:, None], pos30)
        return v, gm30, gid30, pos30

    _, gm30, gid30, pos30 = jax.lax.fori_loop(
        0, TOPK, mrg_body,
        (mv, jnp.full((BQ, 32), NEG, jnp.float32),
         jnp.full((BQ, 32), BIGID, jnp.int32),
         jnp.full((BQ, 32), MW, jnp.int32)))

    src = jnp.concatenate([cand_ref[...], sims3], axis=-1)
    idx = jnp.broadcast_to(pos30[:, None, :TOPK], (BQ, 128, TOPK))
    cand_ref[...] = jnp.take_along_axis(src, idx, axis=2)
    gm_ref[...] = gm30
    gid_ref[...] = gid30

    @pl.when(ck == NKC - 1)
    def _epilogue():
        cand = cand_ref[...]
        l_i = jax.lax.broadcasted_iota(jnp.int32, (BQ, 128, TOPK), 1)
        gidv = gid_ref[...]
        candid = gidv[:, None, :TOPK] * 128 + l_i

        def ext_body(i, carry):
            c, v30, i30 = carry
            m = jnp.max(c, axis=(1, 2))
            ismax = c == m[:, None, None]
            gid = jnp.min(jnp.where(ismax, candid, BIGID), axis=(1, 2))
            hitc = ismax & (candid == gid[:, None, None])
            c = jnp.where(hitc, NEG, c)
            v30 = jnp.where(ji == i, m[:, None], v30)
            i30 = jnp.where(ji == i, gid[:, None], i30)
            return c, v30, i30

        v0 = jnp.zeros((BQ, 32), jnp.float32)
        i0 = jnp.zeros((BQ, 32), jnp.int32)
        _, v30, i30 = jax.lax.fori_loop(0, TOPK, ext_body, (cand, v0, i0))

        vals_ref[...] = jnp.where(v30[:, :TOPK] >= 0.5, v30[:, :TOPK], 0.0)
        ids_ref[...] = i30[:, :TOPK]


@jax.jit
def _run(queries, keys):
    nq = queries.shape[0]
    keys_p = jnp.pad(keys, ((0, K_PAD - K_REAL), (0, 0)))
    keys_p = (keys_p.reshape(NKC, GPC, 128, D)
              .transpose(0, 2, 1, 3).reshape(K_PAD, D))
    vals, ids = pl.pallas_call(
        _topk_kernel,
        grid=(nq // BQ, NKC),
        in_specs=[
            pl.BlockSpec((BQ, D), lambda qi, ck: (qi, 0)),
            pl.BlockSpec((CK, D), lambda qi, ck: (ck, 0)),
        ],
        out_specs=[
            pl.BlockSpec((BQ, TOPK), lambda qi, ck: (qi, 0)),
            pl.BlockSpec((BQ, TOPK), lambda qi, ck: (qi, 0)),
        ],
        out_shape=[
            jax.ShapeDtypeStruct((nq, TOPK), jnp.float32),
            jax.ShapeDtypeStruct((nq, TOPK), jnp.int32),
        ],
        scratch_shapes=[
            pltpu.VMEM((BQ, 128, TOPK), jnp.float32),
            pltpu.VMEM((BQ, 32), jnp.float32),
            pltpu.VMEM((BQ, 32), jnp.int32),
        ],
        compiler_params=pltpu.CompilerParams(
            dimension_semantics=("parallel", "arbitrary")),
    )(queries, keys_p)
    return vals, ids


def kernel(queries, keys, k):
    del k
    return _run(queries, keys)


# speedup vs baseline: 1.6032x; 1.6032x over previous
import jax
import jax.numpy as jnp
from jax.experimental import pallas as pl
from jax.experimental.pallas import tpu as pltpu

K_REAL = 100000
NG = 800
NGP = 896
NC = NGP // 128
K_PAD = 128 * NG
D = 128
BQ = 64
CK = 12800
NKC = K_PAD // CK
RPC = CK // NG
TOPK = 30
NEG = -1e30
BIGID = 2**30


def _topk_kernel(q_ref, k_ref, vals_ref, ids_ref, s_ref, gmax_ref):
    ki = pl.program_id(1)

    q = q_ref[...]
    qn = q / (jnp.sqrt(jnp.sum(q * q, axis=-1, keepdims=True)) + 1e-12)
    sims = jax.lax.dot_general(
        qn, k_ref[...], (((1,), (1,)), ((), ())),
        preferred_element_type=jnp.float32)

    col = ki * CK + jax.lax.broadcasted_iota(jnp.int32, (BQ, CK), 1)
    sims = jnp.where(col < K_REAL, sims, NEG)
    sims3 = sims.reshape(BQ, RPC, NG)
    sims3 = jnp.concatenate(
        [sims3, jnp.full((BQ, RPC, NGP - NG), NEG, jnp.float32)], axis=-1)
    for c in range(NC):
        s_ref[c, :, pl.ds(ki * RPC, RPC), :] = (
            sims3[:, :, c * 128:(c + 1) * 128])

    chunk_max = jnp.max(sims3, axis=1)

    @pl.when(ki == 0)
    def _init_gmax():
        gmax_ref[...] = chunk_max

    @pl.when(ki > 0)
    def _acc_gmax():
        gmax_ref[...] = jnp.maximum(gmax_ref[...], chunk_max)

    @pl.when(ki == NKC - 1)
    def _epilogue():
        gmax = gmax_ref[...]

        def sel_body(i, carry):
            gm, sel = carry
            g = jnp.argmax(gm, axis=-1).astype(jnp.int32)
            lane = jax.lax.broadcasted_iota(jnp.int32, (BQ, NGP), 1)
            gm = jnp.where(lane == g[:, None], NEG, gm)
            ji = jax.lax.broadcasted_iota(jnp.int32, (BQ, 32), 1)
            sel = jnp.where(ji == i, g[:, None], sel)
            return gm, sel

        sel0 = jnp.full((BQ, 32), NGP, dtype=jnp.int32)
        _, sel = jax.lax.fori_loop(0, TOPK, sel_body, (gmax, sel0))
        selg = sel[:, :TOPK]

        idx3 = jnp.broadcast_to(selg[:, None, :], (BQ, 128, TOPK))

        def gat_body(c, cand):
            src = s_ref[c]
            loc = jnp.clip(idx3 - c * 128, 0, 127)
            got = jnp.take_along_axis(src, loc, axis=2)
            valid = (idx3 >= c * 128) & (idx3 < (c + 1) * 128)
            return jnp.where(valid, got, cand)

        cand = jax.lax.fori_loop(
            0, NC, gat_body, jnp.full((BQ, 128, TOPK), NEG, jnp.float32))
        l_i = jax.lax.broadcasted_iota(jnp.int32, (BQ, 128, TOPK), 1)
        candid = l_i * NG + selg[:, None, :]

        def ext_body(i, carry):
            c, v30, i30 = carry
            m = jnp.max(c, axis=(1, 2))
            ismax = c == m[:, None, None]
            gid = jnp.min(jnp.where(ismax, candid, BIGID), axis=(1, 2))
            hit = ismax & (candid == gid[:, None, None])
            c = jnp.where(hit, NEG, c)
            ji = jax.lax.broadcasted_iota(jnp.int32, (BQ, 32), 1)
            v30 = jnp.where(ji == i, m[:, None], v30)
            i30 = jnp.where(ji == i, gid[:, None], i30)
            return c, v30, i30

        v0 = jnp.zeros((BQ, 32), jnp.float32)
        i0 = jnp.zeros((BQ, 32), jnp.int32)
        _, v30, i30 = jax.lax.fori_loop(0, TOPK, ext_body, (cand, v0, i0))

        vals_ref[...] = jnp.where(v30[:, :TOPK] >= 0.5, v30[:, :TOPK], 0.0)
        ids_ref[...] = i30[:, :TOPK]


@jax.jit
def _run(queries, keys):
    nq = queries.shape[0]
    keys_p = jnp.pad(keys, ((0, K_PAD - K_REAL), (0, 0)))
    grid = (nq // BQ, NKC)
    vals, ids = pl.pallas_call(
        _topk_kernel,
        grid=grid,
        in_specs=[
            pl.BlockSpec((BQ, D), lambda qi, ki: (qi, 0)),
            pl.BlockSpec((CK, D), lambda qi, ki: (ki, 0)),
        ],
        out_specs=[
            pl.BlockSpec((BQ, TOPK), lambda qi, ki: (qi, 0)),
            pl.BlockSpec((BQ, TOPK), lambda qi, ki: (qi, 0)),
        ],
        out_shape=[
            jax.ShapeDtypeStruct((nq, TOPK), jnp.float32),
            jax.ShapeDtypeStruct((nq, TOPK), jnp.int32),
        ],
        scratch_shapes=[pltpu.VMEM((NC, BQ, 128, 128), jnp.float32),
                        pltpu.VMEM((BQ, NGP), jnp.float32)],
        compiler_params=pltpu.CompilerParams(
            dimension_semantics=("parallel", "arbitrary")),
    )(queries, keys_p)
    return vals, ids


def kernel(queries, keys, k):
    del k
    return _run(queries, keys)


# speedup vs baseline: 2.9259x; 1.8250x over previous
import jax
import jax.numpy as jnp
from jax.experimental import pallas as pl
from jax.experimental.pallas import tpu as pltpu

K_REAL = 100000
NG = 800
NGP = 896
NC = NGP // 128
K_PAD = 128 * NG
D = 128
BQ = 64
CK = 12800
NKC = K_PAD // CK
RPC = CK // NG
TOPK = 30
NEG = -1e30


def _topk_kernel(q_ref, k_ref, vals_ref, ids_ref, s_ref, gmax_ref):
    ki = pl.program_id(1)

    q = q_ref[...]
    qn = q / (jnp.sqrt(jnp.sum(q * q, axis=-1, keepdims=True)) + 1e-12)
    sims = jax.lax.dot_general(
        qn, k_ref[...], (((1,), (1,)), ((), ())),
        preferred_element_type=jnp.float32)

    col = ki * CK + jax.lax.broadcasted_iota(jnp.int32, (BQ, CK), 1)
    sims = jnp.where(col < K_REAL, sims, NEG)
    sims3 = sims.reshape(BQ, RPC, NG)
    sims3 = jnp.concatenate(
        [sims3, jnp.full((BQ, RPC, NGP - NG), NEG, jnp.float32)], axis=-1)
    for c in range(NC):
        s_ref[c, :, pl.ds(ki * RPC, RPC), :] = (
            sims3[:, :, c * 128:(c + 1) * 128])

    chunk_max = jnp.max(sims3, axis=1)

    @pl.when(ki == 0)
    def _init_gmax():
        gmax_ref[...] = chunk_max

    @pl.when(ki > 0)
    def _acc_gmax():
        gmax_ref[...] = jnp.maximum(gmax_ref[...], chunk_max)

    @pl.when(ki == NKC - 1)
    def _epilogue():
        gmax = gmax_ref[...]

        def sel_body(i, carry):
            gm, sel = carry
            g = jnp.argmax(gm, axis=-1).astype(jnp.int32)
            lane = jax.lax.broadcasted_iota(jnp.int32, (BQ, NGP), 1)
            gm = jnp.where(lane == g[:, None], NEG, gm)
            ji = jax.lax.broadcasted_iota(jnp.int32, (BQ, 32), 1)
            sel = jnp.where(ji == i, g[:, None], sel)
            return gm, sel

        sel0 = jnp.full((BQ, 32), NGP, dtype=jnp.int32)
        _, sel = jax.lax.fori_loop(0, TOPK, sel_body, (gmax, sel0))

        def sort_body(i, carry):
            s_in, s_out = carry
            g = jnp.min(s_in, axis=-1).astype(jnp.int32)
            s_in = jnp.where(s_in == g[:, None], jnp.int32(2 * NGP), s_in)
            lane = jax.lax.broadcasted_iota(jnp.int32, (BQ, 32), 1)
            s_out = jnp.where(lane == i, g[:, None], s_out)
            return s_in, s_out

        _, sels = jax.lax.fori_loop(0, TOPK, sort_body,
                                    (sel, jnp.zeros((BQ, 32), jnp.int32)))
        selg = sels[:, :TOPK]

        idx3 = jnp.broadcast_to(selg[:, None, :], (BQ, 128, TOPK))

        def gat_body(c, cand):
            src = s_ref[c]
            loc = jnp.clip(idx3 - c * 128, 0, 127)
            got = jnp.take_along_axis(src, loc, axis=2)
            valid = (idx3 >= c * 128) & (idx3 < (c + 1) * 128)
            return jnp.where(valid, got, cand)

        cand = jax.lax.fori_loop(
            0, NC, gat_body, jnp.full((BQ, 128, TOPK), NEG, jnp.float32))
        cand = cand.reshape(BQ, 128 * TOPK)

        def ext_body(i, carry):
            c, v30, i30 = carry
            m = jnp.max(c, axis=-1)
            p = jnp.argmax(c, axis=-1).astype(jnp.int32)
            l, j = p // TOPK, p % TOPK
            g = jnp.take_along_axis(selg, j[:, None], axis=-1)
            gid = l[:, None] * NG + g
            lane = jax.lax.broadcasted_iota(jnp.int32, (BQ, 128 * TOPK), 1)
            c = jnp.where(lane == p[:, None], NEG, c)
            ji = jax.lax.broadcasted_iota(jnp.int32, (BQ, 32), 1)
            v30 = jnp.where(ji == i, m[:, None], v30)
            i30 = jnp.where(ji == i, gid, i30)
            return c, v30, i30

        v0 = jnp.zeros((BQ, 32), jnp.float32)
        i0 = jnp.zeros((BQ, 32), jnp.int32)
        _, v30, i30 = jax.lax.fori_loop(0, TOPK, ext_body, (cand, v0, i0))

        vals_ref[...] = jnp.where(v30[:, :TOPK] >= 0.5, v30[:, :TOPK], 0.0)
        ids_ref[...] = i30[:, :TOPK]


@jax.jit
def _run(queries, keys):
    nq = queries.shape[0]
    keys_p = jnp.pad(keys, ((0, K_PAD - K_REAL), (0, 0)))
    grid = (nq // BQ, NKC)
    vals, ids = pl.pallas_call(
        _topk_kernel,
        grid=grid,
        in_specs=[
            pl.BlockSpec((BQ, D), lambda qi, ki: (qi, 0)),
            pl.BlockSpec((CK, D), lambda qi, ki: (ki, 0)),
        ],
        out_specs=[
            pl.BlockSpec((BQ, TOPK), lambda qi, ki: (qi, 0)),
            pl.BlockSpec((BQ, TOPK), lambda qi, ki: (qi, 0)),
        ],
        out_shape=[
            jax.ShapeDtypeStruct((nq, TOPK), jnp.float32),
            jax.ShapeDtypeStruct((nq, TOPK), jnp.int32),
        ],
        scratch_shapes=[pltpu.VMEM((NC, BQ, 128, 128), jnp.float32),
                        pltpu.VMEM((BQ, NGP), jnp.float32)],
        compiler_params=pltpu.CompilerParams(
            dimension_semantics=("parallel", "arbitrary")),
    )(queries, keys_p)
    return vals, ids


def kernel(queries, keys, k):
    del k
    return _run(queries, keys)
